# Initial kernel scaffold; baseline (speedup 1.0000x reference)
import jax
import jax.numpy as jnp
from jax.experimental import pallas as pl


def kernel(x, edge_index, W_embed, b_embed, W1, b1, W2, b2):
    raise NotImplementedError("write your pallas kernel here")


# speedup vs baseline: 1.0594x; 1.0594x over previous
import functools

import jax
import jax.numpy as jnp
from jax import lax
from jax.experimental import pallas as pl
from jax.experimental.pallas import tpu as pltpu
from jax.experimental.pallas import tpu_sc as plsc

N = 10000
E = 320000
C = 128
NC = 2
NS = 16
K = 128
NCHUNK = 80
SCHUNK = 8
EPT = K * NCHUNK
EPAD = EPT * NC * NS
ACC_ROWS = 10240
FR = ACC_ROWS // 8
RPT = ACC_ROWS // NS
FPT = FR // NS
RB = 1000
RB2 = 1024


def _dense_a_body(x_ref, we_ref, be_ref, w1_ref, b1_ref, out_ref):
    h = jnp.dot(x_ref[...], we_ref[...], preferred_element_type=jnp.float32)
    h = h + be_ref[...]
    out_ref[...] = jax.nn.gelu(
        jnp.dot(h, w1_ref[...], preferred_element_type=jnp.float32) + b1_ref[...])


def _dense_b_body(sums_ref, cnt_ref, w2_ref, b2_ref, out_ref):
    p = sums_ref[0] + sums_ref[1]
    mean = p / jnp.maximum(cnt_ref[...], 1.0)
    upd = jax.nn.gelu(jnp.dot(mean, w2_ref[...], preferred_element_type=jnp.float32)
                      + b2_ref[...])
    out_ref[...] = jnp.mean(upd, axis=1, keepdims=True)


def _sc_body(g_hbm, ctab_hbm, src_hbm, dst_hbm, out_sum, out_cnt,
             src_v, dst_v, d8_v, dm_v, buf, cbuf, acc, cac, sem, sem2):
    c = lax.axis_index("c")
    s = lax.axis_index("s")

    zv = jnp.zeros((16,), jnp.float32)

    def zb(i, carry):
        buf[i // (C // 16), pl.ds((i % (C // 16)) * 16, 16)] = zv
        cbuf[i // (C // 16), pl.ds((i % (C // 16)) * 16, 16)] = zv
        return carry

    lax.fori_loop(0, K * (C // 16), zb, 0)

    def za(j, carry):
        pltpu.sync_copy(buf, acc.at[pl.ds(s * RPT + j * K, K), :])
        return carry

    lax.fori_loop(0, RPT // K, za, 0)
    pltpu.sync_copy(cbuf.at[pl.ds(0, FPT), :], cac.at[pl.ds(s * FPT, FPT), :])
    plsc.subcore_barrier()

    def stage(t, carry):
        pltpu.sync_copy(src_hbm.at[c, s, pl.ds(t * SCHUNK, SCHUNK)], src_v)
        pltpu.sync_copy(dst_hbm.at[c, s, pl.ds(t * SCHUNK, SCHUNK)], dst_v)

        def ed(j, carry2):
            def cf(g2, carry3):
                d16 = dst_v[j, pl.ds(g2 * 16, 16)]
                d8_v[pl.ds(g2 * 16, 16)] = lax.shift_right_logical(d16, 3)
                dm_v[pl.ds(g2 * 16, 16)] = lax.bitwise_and(d16, 7)
                return carry3

            lax.fori_loop(0, K // 16, cf, 0)
            cp1 = pltpu.async_copy(g_hbm.at[src_v.at[j]], buf, sem)
            cp2 = pltpu.async_copy(ctab_hbm.at[dm_v], cbuf, sem2)
            cp1.wait()
            cp2.wait()
            pltpu.sync_copy(buf, acc.at[dst_v.at[j]], add=True)
            pltpu.sync_copy(cbuf, cac.at[d8_v], add=True)
            return carry2

        lax.fori_loop(0, SCHUNK, ed, 0)
        return carry

    lax.fori_loop(0, NCHUNK // SCHUNK, stage, 0)
    plsc.subcore_barrier()

    def wout(j, carry):
        r = s * RPT + j * K
        pltpu.sync_copy(acc.at[pl.ds(r, K), :], buf)
        pltpu.sync_copy(buf, out_sum.at[c, pl.ds(r, K), :])
        return carry

    lax.fori_loop(0, RPT // K, wout, 0)
    pltpu.sync_copy(cac.at[pl.ds(s * FPT, FPT), :], cbuf.at[pl.ds(0, FPT), :])
    pltpu.sync_copy(cbuf.at[pl.ds(0, FPT), :], out_cnt.at[c, pl.ds(s * FPT, FPT), :])


def _make_sc_aggregate():
    return functools.partial(
        pl.kernel,
        out_type=(jax.ShapeDtypeStruct((NC, ACC_ROWS, C), jnp.float32),
                  jax.ShapeDtypeStruct((NC, FR, C), jnp.float32)),
        mesh=plsc.VectorSubcoreMesh(core_axis_name="c", subcore_axis_name="s",
                                    num_cores=NC, num_subcores=NS),
        scratch_types=[
            pltpu.VMEM((SCHUNK, K), jnp.int32),
            pltpu.VMEM((SCHUNK, K), jnp.int32),
            pltpu.VMEM((K,), jnp.int32),
            pltpu.VMEM((K,), jnp.int32),
            pltpu.VMEM((K, C), jnp.float32),
            pltpu.VMEM((K, C), jnp.float32),
            pltpu.VMEM_SHARED((ACC_ROWS, C), jnp.float32),
            pltpu.VMEM_SHARED((FR, C), jnp.float32),
            pltpu.SemaphoreType.DMA,
            pltpu.SemaphoreType.DMA,
        ],
    )(_sc_body)


def kernel(x, edge_index, W_embed, b_embed, W1, b1, W2, b2):
    x = x.astype(jnp.float32)

    g = pl.pallas_call(
        _dense_a_body,
        grid=(N // RB,),
        in_specs=[
            pl.BlockSpec((RB, x.shape[1]), lambda i: (i, 0)),
            pl.BlockSpec(W_embed.shape, lambda i: (0, 0)),
            pl.BlockSpec((1, C), lambda i: (0, 0)),
            pl.BlockSpec(W1.shape, lambda i: (0, 0)),
            pl.BlockSpec((1, C), lambda i: (0, 0)),
        ],
        out_specs=pl.BlockSpec((RB, C), lambda i: (i, 0)),
        out_shape=jax.ShapeDtypeStruct((N, C), jnp.float32),
    )(x, W_embed, b_embed.reshape(1, C), W1, b1.reshape(1, C))

    ei = edge_index.astype(jnp.int32)
    pad = EPAD - E
    src_p = jnp.concatenate([ei[0], jnp.zeros((pad,), jnp.int32)])
    dst_p = jnp.concatenate([ei[1], jnp.full((pad,), N, jnp.int32)])
    src_r = src_p.reshape(NC, NS, NCHUNK, K)
    dst_r = dst_p.reshape(NC, NS, NCHUNK, K)

    ctab = jnp.repeat(jnp.eye(8, dtype=jnp.float32), 16, axis=1)

    sums, cnts = _make_sc_aggregate()(g, ctab, src_r, dst_r)

    cnt = cnts.sum(axis=0)[:, ::16].reshape(ACC_ROWS, 1)

    out2d = pl.pallas_call(
        _dense_b_body,
        grid=(ACC_ROWS // RB2,),
        in_specs=[
            pl.BlockSpec((NC, RB2, C), lambda i: (0, i, 0)),
            pl.BlockSpec((RB2, 1), lambda i: (i, 0)),
            pl.BlockSpec(W2.shape, lambda i: (0, 0)),
            pl.BlockSpec((1, C), lambda i: (0, 0)),
        ],
        out_specs=pl.BlockSpec((RB2, 1), lambda i: (i, 0)),
        out_shape=jax.ShapeDtypeStruct((ACC_ROWS, 1), jnp.float32),
    )(sums, cnt, W2, b2.reshape(1, C))

    return out2d.reshape(ACC_ROWS)[:N]


# speedup vs baseline: 3.2443x; 3.0624x over previous
import functools

import jax
import jax.numpy as jnp
from jax import lax
from jax.experimental import pallas as pl
from jax.experimental.pallas import tpu as pltpu
from jax.experimental.pallas import tpu_sc as plsc

N = 10000
E = 320000
C = 128
NC = 2
NS = 16
K = 128
NCHUNK = 80
SCHUNK = 8
EPT = K * NCHUNK
EPAD = EPT * NC * NS
ACC_ROWS = 10240
RPT = ACC_ROWS // NS
RB = 1000
RB2 = 1024


def _dense_a_body(x_ref, we_ref, be_ref, w1_ref, b1_ref, out_ref):
    h = jnp.dot(x_ref[...], we_ref[...], preferred_element_type=jnp.float32)
    h = h + be_ref[...]
    out_ref[...] = jax.nn.gelu(
        jnp.dot(h, w1_ref[...], preferred_element_type=jnp.float32) + b1_ref[...])


def _dense_b_body(sums_ref, cnt_ref, w2_ref, b2_ref, out_ref):
    p = sums_ref[0] + sums_ref[1]
    mean = p / jnp.maximum(cnt_ref[...], 1.0)
    upd = jax.nn.gelu(jnp.dot(mean, w2_ref[...], preferred_element_type=jnp.float32)
                      + b2_ref[...])
    out_ref[...] = jnp.mean(upd, axis=1, keepdims=True)


def _zero_buf(buf):
    zv = jnp.zeros((16,), jnp.float32)

    def zb(i, carry):
        buf[i // (C // 16), pl.ds((i % (C // 16)) * 16, 16)] = zv
        return carry

    lax.fori_loop(0, K * (C // 16), zb, 0)


def _zero_acc_slice(buf, acc, s):
    def za(j, carry):
        pltpu.sync_copy(buf, acc.at[pl.ds(s * RPT + j * K, K), :])
        return carry

    lax.fori_loop(0, RPT // K, za, 0)


def _write_out_slice(buf, acc, out, c, s):
    def wout(j, carry):
        r = s * RPT + j * K
        pltpu.sync_copy(acc.at[pl.ds(r, K), :], buf)
        pltpu.sync_copy(buf, out.at[c, pl.ds(r, K), :])
        return carry

    lax.fori_loop(0, RPT // K, wout, 0)


def _sc_sum_body(g_hbm, src_hbm, dst_hbm, out_sum,
                 src_v, dst_v, buf0, buf1, acc, semg0, semg1, sems0, sems1):
    c = lax.axis_index("c")
    s = lax.axis_index("s")

    _zero_buf(buf0)
    _zero_acc_slice(buf0, acc, s)
    plsc.subcore_barrier()

    bufs = (buf0, buf1)
    semg = (semg0, semg1)
    sems = (sems0, sems1)

    def stage(t, carry):
        pltpu.sync_copy(src_hbm.at[c, s, pl.ds(t * SCHUNK, SCHUNK)], src_v)
        pltpu.sync_copy(dst_hbm.at[c, s, pl.ds(t * SCHUNK, SCHUNK)], dst_v)
        gd = [None, None]
        sd = [None, None]
        for j in range(SCHUNK):
            p = j & 1
            if sd[p] is not None:
                sd[p].wait()
            gd[p] = pltpu.async_copy(g_hbm.at[src_v.at[j]], bufs[p], semg[p])
            if j >= 1:
                q = 1 - p
                gd[q].wait()
                sd[q] = pltpu.async_copy(bufs[q], acc.at[dst_v.at[j - 1]],
                                         sems[q], add=True)
        q = (SCHUNK - 1) & 1
        gd[q].wait()
        sd[q] = pltpu.async_copy(bufs[q], acc.at[dst_v.at[SCHUNK - 1]],
                                 sems[q], add=True)
        for p in range(2):
            if sd[p] is not None:
                sd[p].wait()
        return carry

    lax.fori_loop(0, NCHUNK // SCHUNK, stage, 0)
    plsc.subcore_barrier()
    _write_out_slice(buf0, acc, out_sum, c, s)


def _sc_cnt_body(dst_hbm, out_cnt, dst_v, ones_v, cac, sem):
    c = lax.axis_index("c")
    s = lax.axis_index("s")

    _zero_buf(ones_v)
    _zero_acc_slice(ones_v, cac, s)

    ov = jnp.ones((16,), jnp.float32)

    def fo(i, carry):
        ones_v[i // (C // 16), pl.ds((i % (C // 16)) * 16, 16)] = ov
        return carry

    lax.fori_loop(0, K * (C // 16), fo, 0)
    plsc.subcore_barrier()

    def stage(t, carry):
        pltpu.sync_copy(dst_hbm.at[c, s, pl.ds(t * SCHUNK, SCHUNK)], dst_v)
        sds = [pltpu.async_copy(ones_v, cac.at[dst_v.at[j]], sem, add=True)
               for j in range(SCHUNK)]
        for sd in sds:
            sd.wait()
        return carry

    lax.fori_loop(0, NCHUNK // SCHUNK, stage, 0)
    plsc.subcore_barrier()
    _write_out_slice(ones_v, cac, out_cnt, c, s)


def _make_sc_sum():
    return functools.partial(
        pl.kernel,
        out_type=jax.ShapeDtypeStruct((NC, ACC_ROWS, C), jnp.float32),
        mesh=plsc.VectorSubcoreMesh(core_axis_name="c", subcore_axis_name="s",
                                    num_cores=NC, num_subcores=NS),
        scratch_types=[
            pltpu.VMEM((SCHUNK, K), jnp.int32),
            pltpu.VMEM((SCHUNK, K), jnp.int32),
            pltpu.VMEM((K, C), jnp.float32),
            pltpu.VMEM((K, C), jnp.float32),
            pltpu.VMEM_SHARED((ACC_ROWS, C), jnp.float32),
            pltpu.SemaphoreType.DMA,
            pltpu.SemaphoreType.DMA,
            pltpu.SemaphoreType.DMA,
            pltpu.SemaphoreType.DMA,
        ],
    )(_sc_sum_body)


def _make_sc_cnt():
    return functools.partial(
        pl.kernel,
        out_type=jax.ShapeDtypeStruct((NC, ACC_ROWS, C), jnp.float32),
        mesh=plsc.VectorSubcoreMesh(core_axis_name="c", subcore_axis_name="s",
                                    num_cores=NC, num_subcores=NS),
        scratch_types=[
            pltpu.VMEM((SCHUNK, K), jnp.int32),
            pltpu.VMEM((K, C), jnp.float32),
            pltpu.VMEM_SHARED((ACC_ROWS, C), jnp.float32),
            pltpu.SemaphoreType.DMA,
        ],
    )(_sc_cnt_body)


def kernel(x, edge_index, W_embed, b_embed, W1, b1, W2, b2):
    x = x.astype(jnp.float32)

    g = pl.pallas_call(
        _dense_a_body,
        grid=(N // RB,),
        in_specs=[
            pl.BlockSpec((RB, x.shape[1]), lambda i: (i, 0)),
            pl.BlockSpec(W_embed.shape, lambda i: (0, 0)),
            pl.BlockSpec((1, C), lambda i: (0, 0)),
            pl.BlockSpec(W1.shape, lambda i: (0, 0)),
            pl.BlockSpec((1, C), lambda i: (0, 0)),
        ],
        out_specs=pl.BlockSpec((RB, C), lambda i: (i, 0)),
        out_shape=jax.ShapeDtypeStruct((N, C), jnp.float32),
    )(x, W_embed, b_embed.reshape(1, C), W1, b1.reshape(1, C))

    ei = edge_index.astype(jnp.int32)
    pad = EPAD - E
    src_p = jnp.concatenate([ei[0], jnp.zeros((pad,), jnp.int32)])
    dst_p = jnp.concatenate([ei[1], jnp.full((pad,), N, jnp.int32)])
    src_r = src_p.reshape(NC, NS, NCHUNK, K)
    dst_r = dst_p.reshape(NC, NS, NCHUNK, K)

    sums = _make_sc_sum()(g, src_r, dst_r)
    cnts = _make_sc_cnt()(dst_r)

    cnt = cnts.sum(axis=0)[:, 0:1]

    out2d = pl.pallas_call(
        _dense_b_body,
        grid=(ACC_ROWS // RB2,),
        in_specs=[
            pl.BlockSpec((NC, RB2, C), lambda i: (0, i, 0)),
            pl.BlockSpec((RB2, 1), lambda i: (i, 0)),
            pl.BlockSpec(W2.shape, lambda i: (0, 0)),
            pl.BlockSpec((1, C), lambda i: (0, 0)),
        ],
        out_specs=pl.BlockSpec((RB2, 1), lambda i: (i, 0)),
        out_shape=jax.ShapeDtypeStruct((ACC_ROWS, 1), jnp.float32),
    )(sums, cnt, W2, b2.reshape(1, C))

    return out2d.reshape(ACC_ROWS)[:N]
